# baseline (speedup 1.0000x reference)
import functools

import jax
import jax.numpy as jnp
from jax.experimental import pallas as pl
from jax.experimental.pallas import tpu as pltpu

E = 64
TOPK = 8
LOSS_COEF = 0.01
BM = 1024
BC = 256


def _gating_kernel(x_ref, wt_ref, idx_ref, w_ref, pi_ref, cnt_ref, aux_ref,
                   *, total_tokens):
    i = pl.program_id(0)
    nblk = pl.num_programs(0)

    x = x_ref[...]
    wt = wt_ref[...]
    logits = jnp.dot(x, wt, preferred_element_type=jnp.float32)

    pi_part = jnp.zeros((1, E), jnp.float32)
    cnt_part = jnp.zeros((1, E), jnp.float32)

    for c in range(BM // BC):
        lg = logits[c * BC:(c + 1) * BC, :]
        m = jnp.max(lg, axis=-1, keepdims=True)
        ex = jnp.exp(lg - m)
        denom = jnp.sum(ex, axis=-1, keepdims=True)
        scores = ex / denom

        iota = jax.lax.broadcasted_iota(jnp.int32, scores.shape, 1)
        work = scores
        sel_sum = jnp.zeros_like(scores)
        vals = []
        idxs = []
        for _ in range(TOPK):
            mk = jnp.max(work, axis=-1, keepdims=True)
            is_max = work == mk
            idxk = jnp.min(jnp.where(is_max, iota, E), axis=-1, keepdims=True)
            onehot = iota == idxk
            sel_sum = sel_sum + onehot.astype(jnp.float32)
            work = jnp.where(onehot, -1.0, work)
            vals.append(mk)
            idxs.append(idxk)
        topv = jnp.concatenate(vals, axis=-1)
        topi = jnp.concatenate(idxs, axis=-1)
        topv = topv / jnp.sum(topv, axis=-1, keepdims=True)

        idx_ref[c * BC:(c + 1) * BC, :] = topi.astype(jnp.int32)
        w_ref[c * BC:(c + 1) * BC, :] = topv

        pi_part += jnp.sum(scores, axis=0, keepdims=True)
        cnt_part += jnp.sum(sel_sum, axis=0, keepdims=True)

    @pl.when(i == 0)
    def _init():
        pi_ref[...] = jnp.zeros_like(pi_ref)
        cnt_ref[...] = jnp.zeros_like(cnt_ref)

    pi_ref[...] += pi_part
    cnt_ref[...] += cnt_part

    @pl.when(i == nblk - 1)
    def _finish():
        scale = LOSS_COEF * E / (float(total_tokens) ** 2 * TOPK)
        aux = jnp.sum(pi_ref[...] * cnt_ref[...]) * scale
        aux_ref[...] = jnp.full((1, 1), aux, dtype=jnp.float32)


def kernel(x, W):
    Bd, Nd, R = x.shape
    T = Bd * Nd
    flat_x = x.reshape(T, R)
    wt = W.T

    out_shapes = (
        jax.ShapeDtypeStruct((T, TOPK), jnp.int32),
        jax.ShapeDtypeStruct((T, TOPK), jnp.float32),
        jax.ShapeDtypeStruct((1, E), jnp.float32),
        jax.ShapeDtypeStruct((1, E), jnp.float32),
        jax.ShapeDtypeStruct((1, 1), jnp.float32),
    )
    idx, w, _pi, _cnt, aux = pl.pallas_call(
        functools.partial(_gating_kernel, total_tokens=T),
        grid=(T // BM,),
        in_specs=[
            pl.BlockSpec((BM, R), lambda i: (i, 0)),
            pl.BlockSpec((R, E), lambda i: (0, 0)),
        ],
        out_specs=[
            pl.BlockSpec((BM, TOPK), lambda i: (i, 0)),
            pl.BlockSpec((BM, TOPK), lambda i: (i, 0)),
            pl.BlockSpec((1, E), lambda i: (0, 0)),
            pl.BlockSpec((1, E), lambda i: (0, 0)),
            pl.BlockSpec((1, 1), lambda i: (0, 0)),
        ],
        out_shape=out_shapes,
        compiler_params=pltpu.CompilerParams(
            dimension_semantics=("arbitrary",),
        ),
    )(flat_x, wt)

    return (idx.reshape(Bd, Nd, TOPK), w.reshape(Bd, Nd, TOPK), aux[0, 0])


# speedup vs baseline: 1.1418x; 1.1418x over previous
import functools

import jax
import jax.numpy as jnp
from jax.experimental import pallas as pl
from jax.experimental.pallas import tpu as pltpu

E = 64
TOPK = 8
LOSS_COEF = 0.01
BM = 1024
BC = 256


def _gating_kernel(x_ref, wt_ref, idx_ref, w_ref, pi_ref, cnt_ref, aux_ref,
                   *, total_tokens):
    i = pl.program_id(0)
    nblk = pl.num_programs(0)

    x = x_ref[...]
    wt = wt_ref[...]
    logits = jnp.dot(x, wt, preferred_element_type=jnp.float32)

    pi_part = jnp.zeros((1, E), jnp.float32)
    cnt_part = jnp.zeros((1, E), jnp.float32)

    for c in range(BM // BC):
        lg = logits[c * BC:(c + 1) * BC, :]
        m = jnp.max(lg, axis=-1, keepdims=True)
        ex = jnp.exp(lg - m)
        denom = jnp.sum(ex, axis=-1, keepdims=True)
        scores = ex / denom

        iota = jax.lax.broadcasted_iota(jnp.int32, scores.shape, 1)
        sbits = jax.lax.bitcast_convert_type(scores, jnp.int32)
        work = jax.lax.bitcast_convert_type(
            jnp.bitwise_or(jnp.bitwise_and(sbits, -64), (E - 1) - iota),
            jnp.float32)
        vals = []
        idxs = []
        for _ in range(TOPK):
            mk = jnp.max(work, axis=-1, keepdims=True)
            work = jnp.where(work == mk, -1.0, work)
            mbits = jax.lax.bitcast_convert_type(mk, jnp.int32)
            idxs.append((E - 1) - jnp.bitwise_and(mbits, E - 1))
            vals.append(jax.lax.bitcast_convert_type(
                jnp.bitwise_and(mbits, -64), jnp.float32))
        topv = jnp.concatenate(vals, axis=-1)
        topi = jnp.concatenate(idxs, axis=-1)
        topv = topv / jnp.sum(topv, axis=-1, keepdims=True)

        idx_ref[c * BC:(c + 1) * BC, :] = topi.astype(jnp.int32)
        w_ref[c * BC:(c + 1) * BC, :] = topv

        sel = (work < 0.0).astype(jnp.float32)
        pi_part += jnp.sum(scores, axis=0, keepdims=True)
        cnt_part += jnp.sum(sel, axis=0, keepdims=True)

    @pl.when(i == 0)
    def _init():
        pi_ref[...] = jnp.zeros_like(pi_ref)
        cnt_ref[...] = jnp.zeros_like(cnt_ref)

    pi_ref[...] += pi_part
    cnt_ref[...] += cnt_part

    @pl.when(i == nblk - 1)
    def _finish():
        scale = LOSS_COEF * E / (float(total_tokens) ** 2 * TOPK)
        aux = jnp.sum(pi_ref[...] * cnt_ref[...]) * scale
        aux_ref[...] = jnp.full((1, 1), aux, dtype=jnp.float32)


def kernel(x, W):
    Bd, Nd, R = x.shape
    T = Bd * Nd
    flat_x = x.reshape(T, R)
    wt = W.T

    out_shapes = (
        jax.ShapeDtypeStruct((T, TOPK), jnp.int32),
        jax.ShapeDtypeStruct((T, TOPK), jnp.float32),
        jax.ShapeDtypeStruct((1, E), jnp.float32),
        jax.ShapeDtypeStruct((1, E), jnp.float32),
        jax.ShapeDtypeStruct((1, 1), jnp.float32),
    )
    idx, w, _pi, _cnt, aux = pl.pallas_call(
        functools.partial(_gating_kernel, total_tokens=T),
        grid=(T // BM,),
        in_specs=[
            pl.BlockSpec((BM, R), lambda i: (i, 0)),
            pl.BlockSpec((R, E), lambda i: (0, 0)),
        ],
        out_specs=[
            pl.BlockSpec((BM, TOPK), lambda i: (i, 0)),
            pl.BlockSpec((BM, TOPK), lambda i: (i, 0)),
            pl.BlockSpec((1, E), lambda i: (0, 0)),
            pl.BlockSpec((1, E), lambda i: (0, 0)),
            pl.BlockSpec((1, 1), lambda i: (0, 0)),
        ],
        out_shape=out_shapes,
        compiler_params=pltpu.CompilerParams(
            dimension_semantics=("arbitrary",),
        ),
    )(flat_x, wt)

    return (idx.reshape(Bd, Nd, TOPK), w.reshape(Bd, Nd, TOPK), aux[0, 0])


# speedup vs baseline: 1.3474x; 1.1801x over previous
import functools

import jax
import jax.numpy as jnp
from jax.experimental import pallas as pl
from jax.experimental.pallas import tpu as pltpu

E = 64
TOPK = 8
LOSS_COEF = 0.01
BM = 1024
BC = 256


def _gating_kernel(x_ref, w_ref_in, idx_ref, w_ref, pi_ref, cnt_ref, aux_ref,
                   wt_ref, *, total_tokens):
    i = pl.program_id(0)
    nblk = pl.num_programs(0)

    @pl.when(i == 0)
    def _prep():
        wt_ref[...] = jnp.transpose(w_ref_in[...], (1, 0))
        pi_ref[...] = jnp.zeros_like(pi_ref)
        cnt_ref[...] = jnp.zeros_like(cnt_ref)

    x = x_ref[...]
    wt = wt_ref[...]
    logits = jnp.dot(x, wt, preferred_element_type=jnp.float32)

    pi_part = jnp.zeros((1, E), jnp.float32)
    cnt_part = jnp.zeros((1, E), jnp.float32)

    for c in range(BM // BC):
        lg = logits[c * BC:(c + 1) * BC, :]
        m = jnp.max(lg, axis=-1, keepdims=True)
        ex = jnp.exp(lg - m)
        denom = jnp.sum(ex, axis=-1, keepdims=True)
        scores = ex / denom

        iota = jax.lax.broadcasted_iota(jnp.int32, scores.shape, 1)
        sbits = jax.lax.bitcast_convert_type(scores, jnp.int32)
        work = jax.lax.bitcast_convert_type(
            jnp.bitwise_or(jnp.bitwise_and(sbits, -64), (E - 1) - iota),
            jnp.float32)
        vals = []
        idxs = []
        for _ in range(TOPK):
            mk = jnp.max(work, axis=-1, keepdims=True)
            work = jnp.where(work == mk, -1.0, work)
            mbits = jax.lax.bitcast_convert_type(mk, jnp.int32)
            idxs.append((E - 1) - jnp.bitwise_and(mbits, E - 1))
            vals.append(jax.lax.bitcast_convert_type(
                jnp.bitwise_and(mbits, -64), jnp.float32))
        topv = jnp.concatenate(vals, axis=-1)
        topi = jnp.concatenate(idxs, axis=-1)
        topv = topv / jnp.sum(topv, axis=-1, keepdims=True)

        idx_ref[0, :, c * BC:(c + 1) * BC] = jnp.transpose(
            topi.astype(jnp.int32), (1, 0))
        w_ref[0, :, c * BC:(c + 1) * BC] = jnp.transpose(topv, (1, 0))

        sel = (work < 0.0).astype(jnp.float32)
        pi_part += jnp.sum(scores, axis=0, keepdims=True)
        cnt_part += jnp.sum(sel, axis=0, keepdims=True)

    pi_ref[...] += pi_part
    cnt_ref[...] += cnt_part

    @pl.when(i == nblk - 1)
    def _finish():
        scale = LOSS_COEF * E / (float(total_tokens) ** 2 * TOPK)
        aux = jnp.sum(pi_ref[...] * cnt_ref[...]) * scale
        aux_ref[...] = jnp.full((1, 1), aux, dtype=jnp.float32)


def kernel(x, W):
    Bd, Nd, R = x.shape
    T = Bd * Nd
    flat_x = x.reshape(T, R)
    blocks_per_b = Nd // BM

    out_shapes = (
        jax.ShapeDtypeStruct((Bd, TOPK, Nd), jnp.int32),
        jax.ShapeDtypeStruct((Bd, TOPK, Nd), jnp.float32),
        jax.ShapeDtypeStruct((1, E), jnp.float32),
        jax.ShapeDtypeStruct((1, E), jnp.float32),
        jax.ShapeDtypeStruct((1, 1), jnp.float32),
    )
    idx, w, _pi, _cnt, aux = pl.pallas_call(
        functools.partial(_gating_kernel, total_tokens=T),
        grid=(T // BM,),
        in_specs=[
            pl.BlockSpec((BM, R), lambda i: (i, 0)),
            pl.BlockSpec((E, R), lambda i: (0, 0)),
        ],
        out_specs=[
            pl.BlockSpec((1, TOPK, BM),
                         lambda i: (i // blocks_per_b, 0, i % blocks_per_b)),
            pl.BlockSpec((1, TOPK, BM),
                         lambda i: (i // blocks_per_b, 0, i % blocks_per_b)),
            pl.BlockSpec((1, E), lambda i: (0, 0)),
            pl.BlockSpec((1, E), lambda i: (0, 0)),
            pl.BlockSpec((1, 1), lambda i: (0, 0)),
        ],
        out_shape=out_shapes,
        scratch_shapes=[pltpu.VMEM((R, E), jnp.float32)],
        compiler_params=pltpu.CompilerParams(
            dimension_semantics=("arbitrary",),
        ),
    )(flat_x, W)

    return (jnp.transpose(idx, (0, 2, 1)), jnp.transpose(w, (0, 2, 1)),
            aux[0, 0])


# speedup vs baseline: 1.3495x; 1.0016x over previous
import functools

import jax
import jax.numpy as jnp
from jax.experimental import pallas as pl
from jax.experimental.pallas import tpu as pltpu

E = 64
TOPK = 8
LOSS_COEF = 0.01
BM = 1024
BH = 512
BC = 256


def _gating_kernel(x1_ref, x2_ref, w_ref_in, idx_ref, w_ref, pi_ref, cnt_ref,
                   aux_ref, wt_ref, *, total_tokens):
    i = pl.program_id(0)
    nblk = pl.num_programs(0)

    @pl.when(i == 0)
    def _prep():
        wt_ref[...] = jnp.transpose(w_ref_in[...], (1, 0))
        pi_ref[...] = jnp.zeros_like(pi_ref)
        cnt_ref[...] = jnp.zeros_like(cnt_ref)

    wt = wt_ref[...]
    pi_part = jnp.zeros((1, E), jnp.float32)
    cnt_part = jnp.zeros((1, E), jnp.float32)

    for h, x_ref in enumerate((x1_ref, x2_ref)):
        logits = jnp.dot(x_ref[...], wt,
                         preferred_element_type=jnp.float32)
        for c in range(BH // BC):
            r0 = h * BH + c * BC
            lg = logits[c * BC:(c + 1) * BC, :]
            m = jnp.max(lg, axis=-1, keepdims=True)
            ex = jnp.exp(lg - m)
            denom = jnp.sum(ex, axis=-1, keepdims=True)
            scores = ex / denom

            iota = jax.lax.broadcasted_iota(jnp.int32, scores.shape, 1)
            sbits = jax.lax.bitcast_convert_type(scores, jnp.int32)
            work = jax.lax.bitcast_convert_type(
                jnp.bitwise_or(jnp.bitwise_and(sbits, -64), (E - 1) - iota),
                jnp.float32)
            vals = []
            idxs = []
            for _ in range(TOPK):
                mk = jnp.max(work, axis=-1, keepdims=True)
                work = jnp.where(work == mk, -1.0, work)
                mbits = jax.lax.bitcast_convert_type(mk, jnp.int32)
                idxs.append((E - 1) - jnp.bitwise_and(mbits, E - 1))
                vals.append(jax.lax.bitcast_convert_type(
                    jnp.bitwise_and(mbits, -64), jnp.float32))
            topv = jnp.concatenate(vals, axis=-1)
            topi = jnp.concatenate(idxs, axis=-1)
            topv = topv / jnp.sum(topv, axis=-1, keepdims=True)

            idx_ref[0, :, r0:r0 + BC] = jnp.transpose(
                topi.astype(jnp.int32), (1, 0))
            w_ref[0, :, r0:r0 + BC] = jnp.transpose(topv, (1, 0))

            sel = (work < 0.0).astype(jnp.float32)
            pi_part += jnp.sum(scores, axis=0, keepdims=True)
            cnt_part += jnp.sum(sel, axis=0, keepdims=True)

    pi_ref[...] += pi_part
    cnt_ref[...] += cnt_part

    @pl.when(i == nblk - 1)
    def _finish():
        scale = LOSS_COEF * E / (float(total_tokens) ** 2 * TOPK)
        aux = jnp.sum(pi_ref[...] * cnt_ref[...]) * scale
        aux_ref[...] = jnp.full((1, 1), aux, dtype=jnp.float32)


def kernel(x, W):
    Bd, Nd, R = x.shape
    T = Bd * Nd
    flat_x = x.reshape(T, R)
    blocks_per_b = Nd // BM

    out_shapes = (
        jax.ShapeDtypeStruct((Bd, TOPK, Nd), jnp.int32),
        jax.ShapeDtypeStruct((Bd, TOPK, Nd), jnp.float32),
        jax.ShapeDtypeStruct((1, E), jnp.float32),
        jax.ShapeDtypeStruct((1, E), jnp.float32),
        jax.ShapeDtypeStruct((1, 1), jnp.float32),
    )
    idx, w, _pi, _cnt, aux = pl.pallas_call(
        functools.partial(_gating_kernel, total_tokens=T),
        grid=(T // BM,),
        in_specs=[
            pl.BlockSpec((BH, R), lambda i: (2 * i, 0)),
            pl.BlockSpec((BH, R), lambda i: (2 * i + 1, 0)),
            pl.BlockSpec((E, R), lambda i: (0, 0)),
        ],
        out_specs=[
            pl.BlockSpec((1, TOPK, BM),
                         lambda i: (i // blocks_per_b, 0, i % blocks_per_b)),
            pl.BlockSpec((1, TOPK, BM),
                         lambda i: (i // blocks_per_b, 0, i % blocks_per_b)),
            pl.BlockSpec((1, E), lambda i: (0, 0)),
            pl.BlockSpec((1, E), lambda i: (0, 0)),
            pl.BlockSpec((1, 1), lambda i: (0, 0)),
        ],
        out_shape=out_shapes,
        scratch_shapes=[pltpu.VMEM((R, E), jnp.float32)],
        compiler_params=pltpu.CompilerParams(
            dimension_semantics=("arbitrary",),
        ),
    )(flat_x, flat_x, W)

    return (jnp.transpose(idx, (0, 2, 1)), jnp.transpose(w, (0, 2, 1)),
            aux[0, 0])
